# baseline (speedup 1.0000x reference)
import jax
import jax.numpy as jnp
from jax import lax
from jax.experimental import pallas as pl
from jax.experimental.pallas import tpu as pltpu
from jax.experimental.pallas import tpu_sc as plsc

B_TOTAL = 4096 * 50
EMBED = 128
NW = 32
CHUNK = 160
PER_W = B_TOTAL // NW
N_CHUNKS = PER_W // CHUNK
NBUF = 6


def _embed_kernel(ids_hbm, table_hbm, out_hbm, idx_v, rows_v, *sems):
    gsems, osems = sems[:NBUF], sems[NBUF:]
    wid = lax.axis_index("s") * 2 + lax.axis_index("c")
    base = wid * PER_W
    pltpu.sync_copy(ids_hbm.at[pl.ds(base, PER_W)], idx_v)

    def gather(j, b):
        return pltpu.make_async_copy(
            table_hbm.at[idx_v.at[pl.ds(j * CHUNK, CHUNK)]],
            rows_v.at[b],
            gsems[b],
        )

    def writeback(j, b):
        return pltpu.make_async_copy(
            rows_v.at[b],
            out_hbm.at[pl.ds(base + j * CHUNK, CHUNK)],
            osems[b],
        )

    for j in range(NBUF):
        gather(j, j).start()
    for j in range(N_CHUNKS):
        b = j % NBUF
        gather(j, b).wait()
        writeback(j, b).start()
        if j + NBUF < N_CHUNKS:
            writeback(j, b).wait()
            gather(j + NBUF, b).start()
    for j in range(N_CHUNKS - NBUF, N_CHUNKS):
        writeback(j, j % NBUF).wait()


@jax.jit
def _embed(ids_flat, table):
    mesh = plsc.VectorSubcoreMesh(core_axis_name="c", subcore_axis_name="s")
    return pl.kernel(
        _embed_kernel,
        out_type=jax.ShapeDtypeStruct((B_TOTAL, EMBED), jnp.float32),
        mesh=mesh,
        scratch_types=[
            pltpu.VMEM((PER_W,), jnp.int32),
            pltpu.VMEM((NBUF, CHUNK, EMBED), jnp.float32),
        ]
        + [pltpu.SemaphoreType.DMA] * (2 * NBUF),
    )(ids_flat, table)


def kernel(ids, W):
    batch, seq = ids.shape
    ids_t = ids.T.reshape(B_TOTAL).astype(jnp.int32)
    out = _embed(ids_t, W)
    return out.reshape(seq, batch, EMBED).transpose(1, 0, 2)


# speedup vs baseline: 1.0034x; 1.0034x over previous
import jax
import jax.numpy as jnp
from jax import lax
from jax.experimental import pallas as pl
from jax.experimental.pallas import tpu as pltpu
from jax.experimental.pallas import tpu_sc as plsc

B_TOTAL = 4096 * 50
EMBED = 128
NW = 32
CHUNK = 400
PER_W = B_TOTAL // NW
N_CHUNKS = PER_W // CHUNK
NBUF = 2


def _embed_kernel(ids_hbm, table_hbm, out_hbm, idx_v, rows_v, *sems):
    gsems, osems = sems[:NBUF], sems[NBUF:]
    wid = lax.axis_index("s") * 2 + lax.axis_index("c")
    base = wid * PER_W
    pltpu.sync_copy(ids_hbm.at[pl.ds(base, PER_W)], idx_v)

    def gather(j, b):
        return pltpu.make_async_copy(
            table_hbm.at[idx_v.at[pl.ds(j * CHUNK, CHUNK)]],
            rows_v.at[b],
            gsems[b],
        )

    def writeback(j, b):
        return pltpu.make_async_copy(
            rows_v.at[b],
            out_hbm.at[pl.ds(base + j * CHUNK, CHUNK)],
            osems[b],
        )

    for j in range(NBUF):
        gather(j, j).start()
    for j in range(N_CHUNKS):
        b = j % NBUF
        gather(j, b).wait()
        writeback(j, b).start()
        if j + NBUF < N_CHUNKS:
            writeback(j, b).wait()
            gather(j + NBUF, b).start()
    for j in range(N_CHUNKS - NBUF, N_CHUNKS):
        writeback(j, j % NBUF).wait()


@jax.jit
def _embed(ids_flat, table):
    mesh = plsc.VectorSubcoreMesh(core_axis_name="c", subcore_axis_name="s")
    return pl.kernel(
        _embed_kernel,
        out_type=jax.ShapeDtypeStruct((B_TOTAL, EMBED), jnp.float32),
        mesh=mesh,
        scratch_types=[
            pltpu.VMEM((PER_W,), jnp.int32),
            pltpu.VMEM((NBUF, CHUNK, EMBED), jnp.float32),
        ]
        + [pltpu.SemaphoreType.DMA] * (2 * NBUF),
    )(ids_flat, table)


def kernel(ids, W):
    batch, seq = ids.shape
    ids_t = ids.T.reshape(B_TOTAL).astype(jnp.int32)
    out = _embed(ids_t, W)
    return out.reshape(seq, batch, EMBED).transpose(1, 0, 2)
